# baseline (speedup 1.0000x reference)
import jax
import jax.numpy as jnp
from jax.experimental import pallas as pl
from jax.experimental.pallas import tpu as pltpu

N_TOK = 16384
D_IN = 2048
D_H1 = 1024
D_H2 = 256
D_OUT = 2

TM = 1024


def _mlp_body(
    x_ref, w1_ref, b1_ref, w2_ref, b2_ref, w3_ref, b3_ref, o_ref, w1b_ref, w2b_ref
):
    i = pl.program_id(0)

    @pl.when(i == 0)
    def _pack_weights():
        w1b_ref[...] = w1_ref[...].astype(jnp.bfloat16)
        w2b_ref[...] = w2_ref[...].astype(jnp.bfloat16)

    @pl.when(i > 0)
    def _mlp():
        h = jnp.dot(
            x_ref[...].astype(jnp.bfloat16),
            w1b_ref[...],
            preferred_element_type=jnp.float32,
        )
        h = jnp.maximum(h + b1_ref[...], 0.0)
        h = jnp.dot(
            h.astype(jnp.bfloat16), w2b_ref[...], preferred_element_type=jnp.float32
        )
        h = jnp.maximum(h + b2_ref[...], 0.0)
        o = jnp.dot(h, w3_ref[...], preferred_element_type=jnp.float32)
        o_ref[...] = o + b3_ref[...]


def kernel(input, W1, b1, W2, b2, W3, b3):
    n = input.shape[0]
    nstep = n // TM
    grid = (nstep + 1,)

    def xmap(i):
        return (jnp.maximum(i - 1, 0), 0)

    out = pl.pallas_call(
        _mlp_body,
        grid=grid,
        in_specs=[
            pl.BlockSpec((TM, D_IN), xmap),
            pl.BlockSpec((D_IN, D_H1), lambda i: (0, 0)),
            pl.BlockSpec((1, D_H1), lambda i: (0, 0)),
            pl.BlockSpec((D_H1, D_H2), lambda i: (0, 0)),
            pl.BlockSpec((1, D_H2), lambda i: (0, 0)),
            pl.BlockSpec((D_H2, D_OUT), lambda i: (0, 0)),
            pl.BlockSpec((1, D_OUT), lambda i: (0, 0)),
        ],
        out_specs=pl.BlockSpec((TM, D_OUT), xmap),
        out_shape=jax.ShapeDtypeStruct((n, D_OUT), jnp.float32),
        scratch_shapes=[
            pltpu.VMEM((D_IN, D_H1), jnp.bfloat16),
            pltpu.VMEM((D_H1, D_H2), jnp.bfloat16),
        ],
    )(
        input,
        W1,
        b1.reshape(1, D_H1),
        W2,
        b2.reshape(1, D_H2),
        W3,
        b3.reshape(1, D_OUT),
    )
    return out


# speedup vs baseline: 1.0068x; 1.0068x over previous
import jax
import jax.numpy as jnp
from jax.experimental import pallas as pl

N_TOK = 16384
D_IN = 2048
D_H1 = 1024
D_H2 = 256
D_OUT = 2

TM = 1024


def _mlp_body(x_ref, w1_ref, b1_ref, w2_ref, b2_ref, w3_ref, b3_ref, o_ref):
    h = jnp.dot(x_ref[...], w1_ref[...], preferred_element_type=jnp.float32)
    h = jnp.maximum(h + b1_ref[...][None, :], 0.0)
    h = jnp.dot(h, w2_ref[...], preferred_element_type=jnp.float32)
    h = jnp.maximum(h + b2_ref[...][None, :], 0.0)
    o = jnp.dot(h, w3_ref[...], preferred_element_type=jnp.float32)
    o_ref[...] = o + b3_ref[...][None, :]


def kernel(input, W1, b1, W2, b2, W3, b3):
    n = input.shape[0]
    grid = (n // TM,)

    out = pl.pallas_call(
        _mlp_body,
        grid=grid,
        in_specs=[
            pl.BlockSpec((TM, D_IN), lambda i: (i, 0)),
            pl.BlockSpec((D_IN, D_H1), lambda i: (0, 0)),
            pl.BlockSpec((D_H1,), lambda i: (0,)),
            pl.BlockSpec((D_H1, D_H2), lambda i: (0, 0)),
            pl.BlockSpec((D_H2,), lambda i: (0,)),
            pl.BlockSpec((D_H2, D_OUT), lambda i: (0, 0)),
            pl.BlockSpec((D_OUT,), lambda i: (0,)),
        ],
        out_specs=pl.BlockSpec((TM, D_OUT), lambda i: (i, 0)),
        out_shape=jax.ShapeDtypeStruct((n, D_OUT), jnp.float32),
    )(input, W1, b1, W2, b2, W3, b3)
    return out
